# Initial kernel scaffold; baseline (speedup 1.0000x reference)
import jax
import jax.numpy as jnp
from jax.experimental import pallas as pl


def kernel(x, edge_index, p, Wl1, Wr1, att1, b1, Wl2, Wr2, att2, b2):
    raise NotImplementedError("write your pallas kernel here")


# speedup vs baseline: 13.1536x; 13.1536x over previous
import functools

import jax
import jax.numpy as jnp
from jax import lax
from jax.experimental import pallas as pl
from jax.experimental.pallas import tpu as pltpu
from jax.experimental.pallas import tpu_sc as plsc

NC = 2
NS = 16
NW = NC * NS
CH = 128


def _sc_gather(tables, idxs):
  K = len(tables)
  E = idxs[0].shape[0]
  per = E // NW
  n_main = per // CH
  tail = per - n_main * CH

  out_type = [jax.ShapeDtypeStruct((E, t.shape[1]), jnp.float32)
              for t in tables]
  scratch = []
  for t in tables:
    D = t.shape[1]
    scratch += [
        pltpu.VMEM((CH,), jnp.int32),
        pltpu.VMEM((max(tail, 8),), jnp.int32),
        pltpu.VMEM((CH, D), jnp.float32),
        pltpu.VMEM((max(tail, 8), D), jnp.float32),
        pltpu.SemaphoreType.DMA,
    ]
  mesh = plsc.VectorSubcoreMesh(core_axis_name="c", subcore_axis_name="s")

  @functools.partial(pl.kernel, out_type=out_type, mesh=mesh,
                     scratch_types=scratch)
  def k(*refs):
    tabs = refs[:K]
    idxr = refs[K:2 * K]
    outs = refs[2 * K:3 * K]
    scr = refs[3 * K:]
    wid = lax.axis_index("c") * NS + lax.axis_index("s")
    base = wid * per
    for j in range(K):
      idx_m, idx_t, buf_m, buf_t, sem = scr[5 * j:5 * j + 5]

      def body(i, _, j=j, idx_m=idx_m, buf_m=buf_m, sem=sem):
        off = base + i * CH
        pltpu.sync_copy(idxr[j].at[pl.ds(off, CH)], idx_m)
        pltpu.async_copy(tabs[j].at[idx_m], buf_m, sem).wait()
        pltpu.sync_copy(buf_m, outs[j].at[pl.ds(off, CH)])
        return 0

      lax.fori_loop(0, n_main, body, 0)
      if tail:
        off = base + n_main * CH
        pltpu.sync_copy(idxr[j].at[pl.ds(off, tail)], idx_t)
        pltpu.async_copy(tabs[j].at[idx_t], buf_t, sem).wait()
        pltpu.sync_copy(buf_t, outs[j].at[pl.ds(off, tail)])

  return k(*tables, *idxs)


def _sc_scatter_many(vals64s, idx, n_rows):
  E = idx.shape[0]
  per = E // NW
  n_main = per // CH
  tail = per - n_main * CH
  npad = ((n_rows + NS * 8 - 1) // (NS * 8)) * (NS * 8)
  rpt = npad // NS
  ng = len(vals64s)
  tl = max(tail, 8)
  assert tail in (0, tl)
  CPO = 104
  n_co = rpt // CPO
  co_tail = rpt - n_co * CPO
  assert co_tail % 8 == 0

  out_type = [jax.ShapeDtypeStruct((NC, npad, 64), jnp.float32)
              for _ in range(ng)]
  scratch = [
      pltpu.VMEM((CH,), jnp.int32),
      pltpu.VMEM((tl,), jnp.int32),
      pltpu.VMEM((CH, 64), jnp.float32),
      pltpu.VMEM((tl, 64), jnp.float32),
      pltpu.VMEM((CPO, 64), jnp.float32),
      pltpu.VMEM_SHARED((npad, 64), jnp.float32),
  ]
  mesh = plsc.VectorSubcoreMesh(core_axis_name="c", subcore_axis_name="s")

  @functools.partial(pl.kernel, mesh=mesh, out_type=out_type,
                     scratch_types=scratch)
  def k(*refs):
    vrefs = refs[:ng]
    idxr = refs[ng]
    outs = refs[1 + ng:1 + 2 * ng]
    idx_m, idx_t, val_m, val_t, zbuf, acc = refs[1 + 2 * ng:]
    c = lax.axis_index("c")
    s = lax.axis_index("s")
    base = (c * NS + s) * per

    def zrow(i, _):
      zv = jnp.zeros((16,), jnp.float32)
      for jj in range(4):
        zbuf[i, pl.ds(jj * 16, 16)] = zv
      return 0

    lax.fori_loop(0, CPO, zrow, 0)

    def run(vals, out):
      def zc(kk, _):
        pltpu.sync_copy(zbuf, acc.at[pl.ds(s * rpt + kk * CPO, CPO)])
        return 0

      lax.fori_loop(0, n_co, zc, 0)
      if co_tail:
        pltpu.sync_copy(zbuf.at[pl.ds(0, co_tail)],
                        acc.at[pl.ds(s * rpt + n_co * CPO, co_tail)])
      plsc.subcore_barrier()

      def body(i, _):
        off = base + i * CH
        pltpu.sync_copy(idxr.at[pl.ds(off, CH)], idx_m)
        pltpu.sync_copy(vals.at[pl.ds(off, CH)], val_m)
        pltpu.sync_copy(val_m, acc.at[idx_m], add=True)
        return 0

      lax.fori_loop(0, n_main, body, 0)
      if tail:
        off = base + n_main * CH
        pltpu.sync_copy(idxr.at[pl.ds(off, tail)], idx_t)
        pltpu.sync_copy(vals.at[pl.ds(off, tail)], val_t)
        pltpu.sync_copy(val_t, acc.at[idx_t], add=True)
      plsc.subcore_barrier()

      def oc(kk, _):
        pltpu.sync_copy(acc.at[pl.ds(s * rpt + kk * CPO, CPO)],
                        out.at[c].at[pl.ds(s * rpt + kk * CPO, CPO)])
        return 0

      lax.fori_loop(0, n_co, oc, 0)
      if co_tail:
        pltpu.sync_copy(acc.at[pl.ds(s * rpt + n_co * CPO, co_tail)],
                        out.at[c].at[pl.ds(s * rpt + n_co * CPO, co_tail)])
      plsc.subcore_barrier()

    for g in range(ng):
      run(vrefs[g], outs[g])

  return list(k(*vals64s, idx))


def _tc_call(body, grid, in_specs, out_specs, out_shapes):
  return pl.pallas_call(
      body, grid=grid, in_specs=in_specs, out_specs=out_specs,
      out_shape=out_shapes)


def _mm2(x, wa, wb):
  n, kdim = x.shape
  da, db = wa.shape[1], wb.shape[1]
  R = 1000

  def body(x_ref, wa_ref, wb_ref, oa_ref, ob_ref):
    xv = x_ref[...]
    oa_ref[...] = jnp.dot(xv, wa_ref[...], preferred_element_type=jnp.float32)
    ob_ref[...] = jnp.dot(xv, wb_ref[...], preferred_element_type=jnp.float32)

  return _tc_call(
      body, (n // R,),
      [pl.BlockSpec((R, kdim), lambda i: (i, 0)),
       pl.BlockSpec((kdim, da), lambda i: (0, 0)),
       pl.BlockSpec((kdim, db), lambda i: (0, 0))],
      [pl.BlockSpec((R, da), lambda i: (i, 0)),
       pl.BlockSpec((R, db), lambda i: (i, 0))],
      [jax.ShapeDtypeStruct((n, da), jnp.float32),
       jax.ShapeDtypeStruct((n, db), jnp.float32)])(x, wa, wb)


def _edge_exp_logits1(rows_l, rows_r, att_flat, heads, ch):
  E, D = rows_l.shape
  B = 2000

  def body(l_ref, r_ref, a_ref, o_ref):
    z = l_ref[...] + r_ref[...]
    z = jnp.where(z >= 0, z, 0.2 * z)
    zw = z * a_ref[...]
    ii = lax.broadcasted_iota(jnp.int32, (D, 16), 0)
    jj = lax.broadcasted_iota(jnp.int32, (D, 16), 1)
    sel = ((ii // ch) == jj).astype(jnp.float32)
    o_ref[...] = jnp.exp(jnp.dot(zw, sel, preferred_element_type=jnp.float32))

  return _tc_call(
      body, (E // B,),
      [pl.BlockSpec((B, D), lambda i: (i, 0)),
       pl.BlockSpec((B, D), lambda i: (i, 0)),
       pl.BlockSpec((1, D), lambda i: (0, 0))],
      pl.BlockSpec((B, 16), lambda i: (i, 0)),
      jax.ShapeDtypeStruct((E, 16), jnp.float32))(rows_l, rows_r, att_flat)


def _edge_msg1(rows_l, s, heads, ch):
  E, D = rows_l.shape
  B = 2000
  H = D // 2

  H = 64

  def body(l_ref, s_ref, *outs):
    sv = s_ref[...]
    sh = sv[:, :heads]
    ii = lax.broadcasted_iota(jnp.int32, (heads, D), 0)
    jj = lax.broadcasted_iota(jnp.int32, (heads, D), 1)
    sel = (ii == (jj // ch)).astype(jnp.float32)
    msg = l_ref[...] * jnp.dot(sh, sel, preferred_element_type=jnp.float32)
    for g in range(D // H):
      outs[g][...] = msg[:, g * H:(g + 1) * H]
    i2 = lax.broadcasted_iota(jnp.int32, (16, H), 0)
    j2 = lax.broadcasted_iota(jnp.int32, (16, H), 1)
    pad = (i2 == j2).astype(jnp.float32)
    outs[-1][...] = jnp.dot(sv, pad, preferred_element_type=jnp.float32)

  ng = D // H
  return _tc_call(
      body, (E // B,),
      [pl.BlockSpec((B, D), lambda i: (i, 0)),
       pl.BlockSpec((B, 16), lambda i: (i, 0))],
      [pl.BlockSpec((B, H), lambda i: (i, 0)) for _ in range(ng + 1)],
      [jax.ShapeDtypeStruct((E, H), jnp.float32) for _ in range(ng + 1)])(
          rows_l, s)


def _combine_elu_mm(groups, den, b1, wl2, wr2, n, heads, ch):
  H = groups[0].shape[2]
  D = H * len(groups)
  dout = wl2.shape[1]
  R = 1000

  def body(*refs):
    grefs = refs[:2 * len(groups)]
    dn0, dn1, bias, wl, wr, ol, orr = refs[2 * len(groups):]
    u = jnp.concatenate(
        [grefs[2 * g][0] + grefs[2 * g + 1][0] for g in range(len(groups))],
        axis=1)
    den8 = dn0[0][:, :heads] + dn1[0][:, :heads] + 1e-16
    ii = lax.broadcasted_iota(jnp.int32, (heads, D), 0)
    jj = lax.broadcasted_iota(jnp.int32, (heads, D), 1)
    sel = (ii == (jj // ch)).astype(jnp.float32)
    h = u * jnp.dot(1.0 / den8, sel, preferred_element_type=jnp.float32)
    h = h + bias[...]
    h = jnp.where(h > 0, h, jnp.exp(jnp.minimum(h, 0.0)) - 1.0)
    ol[...] = jnp.dot(h, wl[...], preferred_element_type=jnp.float32)
    orr[...] = jnp.dot(h, wr[...], preferred_element_type=jnp.float32)

  gspecs = []
  gargs = []
  for g in groups:
    gspecs += [pl.BlockSpec((1, R, H), lambda i: (0, i, 0)),
               pl.BlockSpec((1, R, H), lambda i: (1, i, 0))]
    gargs += [g, g]
  return _tc_call(
      body, (n // R,),
      gspecs + [
          pl.BlockSpec((1, R, 64), lambda i: (0, i, 0)),
          pl.BlockSpec((1, R, 64), lambda i: (1, i, 0)),
          pl.BlockSpec((1, D), lambda i: (0, 0)),
          pl.BlockSpec((D, dout), lambda i: (0, 0)),
          pl.BlockSpec((D, dout), lambda i: (0, 0))],
      [pl.BlockSpec((R, dout), lambda i: (i, 0)),
       pl.BlockSpec((R, dout), lambda i: (i, 0))],
      [jax.ShapeDtypeStruct((n, dout), jnp.float32),
       jax.ShapeDtypeStruct((n, dout), jnp.float32)])(
          *gargs, den, den, b1, wl2, wr2)


def _edge_exp_logits2(rows_l, rows_r, att_row):
  E, D = rows_l.shape
  B = 2000

  def body(l_ref, r_ref, a_ref, o_ref):
    z = l_ref[...] + r_ref[...]
    z = jnp.where(z >= 0, z, 0.2 * z)
    lg = jnp.sum(z * a_ref[...], axis=1, keepdims=True)
    o_ref[...] = jnp.broadcast_to(jnp.exp(lg), (B, 16))

  return _tc_call(
      body, (E // B,),
      [pl.BlockSpec((B, D), lambda i: (i, 0)),
       pl.BlockSpec((B, D), lambda i: (i, 0)),
       pl.BlockSpec((1, D), lambda i: (0, 0))],
      pl.BlockSpec((B, 16), lambda i: (i, 0)),
      jax.ShapeDtypeStruct((E, 16), jnp.float32))(rows_l, rows_r, att_row)


def _edge_msg2(rows_l, s, dout):
  E, D = rows_l.shape
  B = 2000

  def body(l_ref, s_ref, o_ref, sp_ref):
    sv = s_ref[...]
    o_ref[...] = l_ref[...][:, :dout] * sv[:, :1]
    i2 = lax.broadcasted_iota(jnp.int32, (16, 64), 0)
    j2 = lax.broadcasted_iota(jnp.int32, (16, 64), 1)
    pad = (i2 == j2).astype(jnp.float32)
    sp_ref[...] = jnp.dot(sv, pad, preferred_element_type=jnp.float32)

  return _tc_call(
      body, (E // B,),
      [pl.BlockSpec((B, D), lambda i: (i, 0)),
       pl.BlockSpec((B, 16), lambda i: (i, 0))],
      [pl.BlockSpec((B, dout), lambda i: (i, 0)),
       pl.BlockSpec((B, 64), lambda i: (i, 0))],
      [jax.ShapeDtypeStruct((E, dout), jnp.float32),
       jax.ShapeDtypeStruct((E, 64), jnp.float32)])(rows_l, s)


def _final_logsoftmax(q, den, b2, n, dout):
  Dp = q.shape[2]
  R = 1000

  def body(a_ref, b_ref, dn0, dn1, bias, o_ref):
    o = (a_ref[0][:, :dout] + b_ref[0][:, :dout]) / (
        dn0[0][:, :1] + dn1[0][:, :1] + 1e-16) + bias[...]
    m = jnp.max(o, axis=1, keepdims=True)
    lse = m + jnp.log(jnp.sum(jnp.exp(o - m), axis=1, keepdims=True))
    o_ref[...] = o - lse

  return _tc_call(
      body, (n // R,),
      [pl.BlockSpec((1, R, Dp), lambda i: (0, i, 0)),
       pl.BlockSpec((1, R, Dp), lambda i: (1, i, 0)),
       pl.BlockSpec((1, R, 64), lambda i: (0, i, 0)),
       pl.BlockSpec((1, R, 64), lambda i: (1, i, 0)),
       pl.BlockSpec((1, dout), lambda i: (0, 0))],
      pl.BlockSpec((R, dout), lambda i: (i, 0)),
      jax.ShapeDtypeStruct((n, dout), jnp.float32))(q, q, den, den, b2)


def kernel(x, edge_index, p, Wl1, Wr1, att1, b1, Wl2, Wr2, att2, b2):
  n = x.shape[0]
  heads, ch = att1.shape
  src = edge_index[0].astype(jnp.int32)
  dst = edge_index[1].astype(jnp.int32)

  dout = Wl2.shape[1]

  xl1, xr1 = _mm2(x, Wl1, Wr1)
  rows_l, rows_r = _sc_gather([xl1, xr1], [src, dst])
  s1 = _edge_exp_logits1(rows_l, rows_r, att1.reshape(1, heads * ch),
                         heads, ch)
  msgs = _edge_msg1(rows_l, s1, heads, ch)
  parts = _sc_scatter_many(msgs, dst, n)
  groups, den1 = parts[:-1], parts[-1]

  wl2p = jnp.pad(Wl2, ((0, 0), (0, 128 - dout)))
  wr2p = jnp.pad(Wr2, ((0, 0), (0, 128 - dout)))
  att2p = jnp.pad(att2.reshape(1, -1), ((0, 0), (0, 128 - dout)))
  xl2, xr2 = _combine_elu_mm(groups, den1, b1.reshape(1, -1), wl2p, wr2p,
                             n, heads, ch)

  rows_l2, rows_r2 = _sc_gather([xl2, xr2], [src, dst])
  s2 = _edge_exp_logits2(rows_l2, rows_r2, att2p)
  msg2, s2p = _edge_msg2(rows_l2, s2, dout)
  q, den2 = _sc_scatter_many([msg2, s2p], dst, n)

  return _final_logsoftmax(q, den2, b2.reshape(1, -1), n, dout)


# speedup vs baseline: 16.9691x; 1.2901x over previous
import functools

import jax
import jax.numpy as jnp
from jax import lax
from jax.experimental import pallas as pl
from jax.experimental.pallas import tpu as pltpu
from jax.experimental.pallas import tpu_sc as plsc

NC = 2
NS = 16
NW = NC * NS
CH = 128


def _sc_gather(tables, idxs):
  K = len(tables)
  E = idxs[0].shape[0]
  per = E // NW
  n_main = per // CH
  tail = per - n_main * CH

  tl = max(tail, 8)
  D = tables[0].shape[1]
  assert all(t.shape[1] == D for t in tables)
  out_type = [jax.ShapeDtypeStruct((E, D), jnp.float32) for t in tables]
  scratch = [
      pltpu.VMEM((CH, D), jnp.float32),
      pltpu.VMEM((CH, D), jnp.float32),
      pltpu.VMEM((tl, D), jnp.float32),
      pltpu.VMEM((CH,), jnp.int32),
      pltpu.VMEM((CH,), jnp.int32),
      pltpu.VMEM((tl,), jnp.int32),
      pltpu.SemaphoreType.DMA,
      pltpu.SemaphoreType.DMA,
      pltpu.SemaphoreType.DMA,
      pltpu.SemaphoreType.DMA,
  ]
  mesh = plsc.VectorSubcoreMesh(core_axis_name="c", subcore_axis_name="s")

  @functools.partial(pl.kernel, out_type=out_type, mesh=mesh,
                     scratch_types=scratch)
  def k(*refs):
    tabs = refs[:K]
    idxr = refs[K:2 * K]
    outs = refs[2 * K:3 * K]
    (buf_a, buf_b, buf_t, idx_a, idx_b, idx_t, sem_a, sem_b, sem_ix,
     sem_st) = refs[3 * K:]
    bufs = (buf_a, buf_b)
    idxs_v = (idx_a, idx_b)
    sems = (sem_a, sem_b)
    wid = lax.axis_index("c") * NS + lax.axis_index("s")
    base = wid * per
    n_pairs = n_main // 2
    assert n_main == 2 * n_pairs + 1
    last = n_main - 1
    for j in range(K):
      tab, idxj, outj = tabs[j], idxr[j], outs[j]

      def gfire(i, b, tab=tab, idxj=idxj):
        d = pltpu.make_async_copy(idxj.at[pl.ds(base + i * CH, CH)],
                                  idxs_v[b], sem_ix)
        d.start()
        d.wait()
        pltpu.async_copy(tab.at[idxs_v[b]], bufs[b], sems[b])

      def gwait(b, tab=tab):
        pltpu.make_async_copy(tab.at[idxs_v[b]], bufs[b], sems[b]).wait()

      def gstore(i, b, outj=outj):
        d = pltpu.make_async_copy(bufs[b],
                                  outj.at[pl.ds(base + i * CH, CH)], sem_st)
        d.start()
        d.wait()

      gfire(0, 0)
      gfire(1, 1)

      def pair(t, _, gfire=gfire, gwait=gwait, gstore=gstore):
        ia = 2 * t
        gwait(0)
        gstore(ia, 0)
        gfire(ia + 2, 0)
        ib = ia + 1
        gwait(1)
        gstore(ib, 1)
        gfire(jnp.minimum(ib + 2, last), 1)
        return 0

      lax.fori_loop(0, n_pairs, pair, 0)
      gwait(0)
      gstore(last, 0)
      gwait(1)
      if tail:
        off = base + n_main * CH
        d = pltpu.make_async_copy(idxj.at[pl.ds(off, tail)], idx_t, sem_ix)
        d.start()
        d.wait()
        pltpu.async_copy(tab.at[idx_t], buf_t, sem_b).wait()
        d = pltpu.make_async_copy(buf_t, outj.at[pl.ds(off, tail)], sem_st)
        d.start()
        d.wait()

  return k(*tables, *idxs)


def _sc_scatter_many(vals64s, idx, n_rows):
  E = idx.shape[0]
  per = E // NW
  n_main = per // CH
  tail = per - n_main * CH
  npad = ((n_rows + NS * 8 - 1) // (NS * 8)) * (NS * 8)
  rpt = npad // NS
  ng = len(vals64s)
  tl = max(tail, 8)
  assert tail in (0, tl)
  CPO = 104
  n_co = rpt // CPO
  co_tail = rpt - n_co * CPO
  assert co_tail % 8 == 0

  out_type = [jax.ShapeDtypeStruct((NC, npad, 64), jnp.float32)
              for _ in range(ng)]
  scratch = [
      pltpu.VMEM((CH,), jnp.int32),
      pltpu.VMEM((CH,), jnp.int32),
      pltpu.VMEM((tl,), jnp.int32),
      pltpu.VMEM((CH, 64), jnp.float32),
      pltpu.VMEM((CH, 64), jnp.float32),
      pltpu.VMEM((tl, 64), jnp.float32),
      pltpu.VMEM((CPO, 64), jnp.float32),
      pltpu.VMEM_SHARED((npad, 64), jnp.float32),
      pltpu.SemaphoreType.DMA,
      pltpu.SemaphoreType.DMA,
      pltpu.SemaphoreType.DMA,
      pltpu.SemaphoreType.DMA,
  ]
  mesh = plsc.VectorSubcoreMesh(core_axis_name="c", subcore_axis_name="s")

  @functools.partial(pl.kernel, mesh=mesh, out_type=out_type,
                     scratch_types=scratch)
  def k(*refs):
    vrefs = refs[:ng]
    idxr = refs[ng]
    outs = refs[1 + ng:1 + 2 * ng]
    (idx_ma, idx_mb, idx_t, val_a, val_b, val_t, zbuf, acc,
     sem_a, sem_b, sem_ix, sem_sc) = refs[1 + 2 * ng:]
    idx_ms = (idx_ma, idx_mb)
    vbufs = (val_a, val_b)
    sems = (sem_a, sem_b)
    c = lax.axis_index("c")
    s = lax.axis_index("s")
    base = (c * NS + s) * per

    def zrow(i, _):
      zv = jnp.zeros((16,), jnp.float32)
      for jj in range(4):
        zbuf[i, pl.ds(jj * 16, 16)] = zv
      return 0

    lax.fori_loop(0, CPO, zrow, 0)

    n_pairs = n_main // 2
    assert n_main == 2 * n_pairs + 1

    def run(vals, out):
      def zc(kk, _):
        pltpu.sync_copy(zbuf, acc.at[pl.ds(s * rpt + kk * CPO, CPO)])
        return 0

      lax.fori_loop(0, n_co, zc, 0)
      if co_tail:
        pltpu.sync_copy(zbuf.at[pl.ds(0, co_tail)],
                        acc.at[pl.ds(s * rpt + n_co * CPO, co_tail)])
      plsc.subcore_barrier()

      def vfire(i, b, vals=vals):
        pltpu.async_copy(vals.at[pl.ds(base + i * CH, CH)],
                         vbufs[b], sems[b])

      def vwait(i, b, vals=vals):
        pltpu.make_async_copy(vals.at[pl.ds(base + i * CH, CH)],
                              vbufs[b], sems[b]).wait()

      vfire(0, 0)
      vfire(1, 1)

      def scat(i, b):
        d = pltpu.make_async_copy(idxr.at[pl.ds(base + i * CH, CH)],
                                  idx_ms[b], sem_ix)
        d.start()
        d.wait()
        d = pltpu.make_async_copy(vbufs[b], acc.at[idx_ms[b]], sem_sc)
        d.start(add=True)
        d.wait()

      last = n_main - 1

      def pair(t, _, vfire=vfire, vwait=vwait, scat=scat):
        ia = 2 * t
        vwait(ia, 0)
        scat(ia, 0)
        vfire(ia + 2, 0)
        ib = ia + 1
        vwait(ib, 1)
        scat(ib, 1)
        vfire(jnp.minimum(ib + 2, last), 1)
        return 0

      lax.fori_loop(0, n_pairs, pair, 0)
      vwait(last, 0)
      scat(last, 0)
      vwait(last, 1)
      if tail:
        off = base + n_main * CH
        d = pltpu.make_async_copy(idxr.at[pl.ds(off, tail)], idx_t, sem_ix)
        d.start()
        d.wait()
        d = pltpu.make_async_copy(vals.at[pl.ds(off, tail)], val_t, sem_sc)
        d.start()
        d.wait()
        d = pltpu.make_async_copy(val_t, acc.at[idx_t], sem_sc)
        d.start(add=True)
        d.wait()
      plsc.subcore_barrier()

      def oc(kk, _):
        pltpu.sync_copy(acc.at[pl.ds(s * rpt + kk * CPO, CPO)],
                        out.at[c].at[pl.ds(s * rpt + kk * CPO, CPO)])
        return 0

      lax.fori_loop(0, n_co, oc, 0)
      if co_tail:
        pltpu.sync_copy(acc.at[pl.ds(s * rpt + n_co * CPO, co_tail)],
                        out.at[c].at[pl.ds(s * rpt + n_co * CPO, co_tail)])
      plsc.subcore_barrier()

    for g in range(ng):
      run(vrefs[g], outs[g])

  return list(k(*vals64s, idx))


def _tc_call(body, grid, in_specs, out_specs, out_shapes):
  return pl.pallas_call(
      body, grid=grid, in_specs=in_specs, out_specs=out_specs,
      out_shape=out_shapes)


def _mm2(x, wa, wb):
  n, kdim = x.shape
  da, db = wa.shape[1], wb.shape[1]
  R = 1000

  def body(x_ref, wa_ref, wb_ref, oa_ref, ob_ref):
    xv = x_ref[...]
    oa_ref[...] = jnp.dot(xv, wa_ref[...], preferred_element_type=jnp.float32)
    ob_ref[...] = jnp.dot(xv, wb_ref[...], preferred_element_type=jnp.float32)

  return _tc_call(
      body, (n // R,),
      [pl.BlockSpec((R, kdim), lambda i: (i, 0)),
       pl.BlockSpec((kdim, da), lambda i: (0, 0)),
       pl.BlockSpec((kdim, db), lambda i: (0, 0))],
      [pl.BlockSpec((R, da), lambda i: (i, 0)),
       pl.BlockSpec((R, db), lambda i: (i, 0))],
      [jax.ShapeDtypeStruct((n, da), jnp.float32),
       jax.ShapeDtypeStruct((n, db), jnp.float32)])(x, wa, wb)


def _edge_exp_logits1(rows_l, rows_r, att_flat, heads, ch):
  E, D = rows_l.shape
  B = 2000

  def body(l_ref, r_ref, a_ref, o_ref):
    z = l_ref[...] + r_ref[...]
    z = jnp.where(z >= 0, z, 0.2 * z)
    zw = z * a_ref[...]
    ii = lax.broadcasted_iota(jnp.int32, (D, 16), 0)
    jj = lax.broadcasted_iota(jnp.int32, (D, 16), 1)
    sel = ((ii // ch) == jj).astype(jnp.float32)
    o_ref[...] = jnp.exp(jnp.dot(zw, sel, preferred_element_type=jnp.float32))

  return _tc_call(
      body, (E // B,),
      [pl.BlockSpec((B, D), lambda i: (i, 0)),
       pl.BlockSpec((B, D), lambda i: (i, 0)),
       pl.BlockSpec((1, D), lambda i: (0, 0))],
      pl.BlockSpec((B, 16), lambda i: (i, 0)),
      jax.ShapeDtypeStruct((E, 16), jnp.float32))(rows_l, rows_r, att_flat)


def _edge_msg1(rows_l, s, heads, ch):
  E, D = rows_l.shape
  B = 2000
  H = D // 2

  H = 64

  def body(l_ref, s_ref, *outs):
    sv = s_ref[...]
    sh = sv[:, :heads]
    ii = lax.broadcasted_iota(jnp.int32, (heads, D), 0)
    jj = lax.broadcasted_iota(jnp.int32, (heads, D), 1)
    sel = (ii == (jj // ch)).astype(jnp.float32)
    msg = l_ref[...] * jnp.dot(sh, sel, preferred_element_type=jnp.float32)
    for g in range(D // H):
      outs[g][...] = msg[:, g * H:(g + 1) * H]
    i2 = lax.broadcasted_iota(jnp.int32, (16, H), 0)
    j2 = lax.broadcasted_iota(jnp.int32, (16, H), 1)
    pad = (i2 == j2).astype(jnp.float32)
    outs[-1][...] = jnp.dot(sv, pad, preferred_element_type=jnp.float32)

  ng = D // H
  return _tc_call(
      body, (E // B,),
      [pl.BlockSpec((B, D), lambda i: (i, 0)),
       pl.BlockSpec((B, 16), lambda i: (i, 0))],
      [pl.BlockSpec((B, H), lambda i: (i, 0)) for _ in range(ng + 1)],
      [jax.ShapeDtypeStruct((E, H), jnp.float32) for _ in range(ng + 1)])(
          rows_l, s)


def _combine_elu_mm(groups, den, b1, wl2, wr2, n, heads, ch):
  H = groups[0].shape[2]
  D = H * len(groups)
  dout = wl2.shape[1]
  R = 1000

  def body(*refs):
    grefs = refs[:2 * len(groups)]
    dn0, dn1, bias, wl, wr, ol, orr = refs[2 * len(groups):]
    u = jnp.concatenate(
        [grefs[2 * g][0] + grefs[2 * g + 1][0] for g in range(len(groups))],
        axis=1)
    den8 = dn0[0][:, :heads] + dn1[0][:, :heads] + 1e-16
    ii = lax.broadcasted_iota(jnp.int32, (heads, D), 0)
    jj = lax.broadcasted_iota(jnp.int32, (heads, D), 1)
    sel = (ii == (jj // ch)).astype(jnp.float32)
    h = u * jnp.dot(1.0 / den8, sel, preferred_element_type=jnp.float32)
    h = h + bias[...]
    h = jnp.where(h > 0, h, jnp.exp(jnp.minimum(h, 0.0)) - 1.0)
    ol[...] = jnp.dot(h, wl[...], preferred_element_type=jnp.float32)
    orr[...] = jnp.dot(h, wr[...], preferred_element_type=jnp.float32)

  gspecs = []
  gargs = []
  for g in groups:
    gspecs += [pl.BlockSpec((1, R, H), lambda i: (0, i, 0)),
               pl.BlockSpec((1, R, H), lambda i: (1, i, 0))]
    gargs += [g, g]
  return _tc_call(
      body, (n // R,),
      gspecs + [
          pl.BlockSpec((1, R, 64), lambda i: (0, i, 0)),
          pl.BlockSpec((1, R, 64), lambda i: (1, i, 0)),
          pl.BlockSpec((1, D), lambda i: (0, 0)),
          pl.BlockSpec((D, dout), lambda i: (0, 0)),
          pl.BlockSpec((D, dout), lambda i: (0, 0))],
      [pl.BlockSpec((R, dout), lambda i: (i, 0)),
       pl.BlockSpec((R, dout), lambda i: (i, 0))],
      [jax.ShapeDtypeStruct((n, dout), jnp.float32),
       jax.ShapeDtypeStruct((n, dout), jnp.float32)])(
          *gargs, den, den, b1, wl2, wr2)


def _edge_exp_logits2(rows_l, rows_r, att_row):
  E, D = rows_l.shape
  B = 2000

  def body(l_ref, r_ref, a_ref, o_ref):
    z = l_ref[...] + r_ref[...]
    z = jnp.where(z >= 0, z, 0.2 * z)
    lg = jnp.sum(z * a_ref[...], axis=1, keepdims=True)
    o_ref[...] = jnp.broadcast_to(jnp.exp(lg), (B, 16))

  return _tc_call(
      body, (E // B,),
      [pl.BlockSpec((B, D), lambda i: (i, 0)),
       pl.BlockSpec((B, D), lambda i: (i, 0)),
       pl.BlockSpec((1, D), lambda i: (0, 0))],
      pl.BlockSpec((B, 16), lambda i: (i, 0)),
      jax.ShapeDtypeStruct((E, 16), jnp.float32))(rows_l, rows_r, att_row)


def _edge_msg2(rows_l, s, dout):
  E, D = rows_l.shape
  B = 2000

  def body(l_ref, s_ref, o_ref, sp_ref):
    sv = s_ref[...]
    o_ref[...] = l_ref[...][:, :dout] * sv[:, :1]
    i2 = lax.broadcasted_iota(jnp.int32, (16, 64), 0)
    j2 = lax.broadcasted_iota(jnp.int32, (16, 64), 1)
    pad = (i2 == j2).astype(jnp.float32)
    sp_ref[...] = jnp.dot(sv, pad, preferred_element_type=jnp.float32)

  return _tc_call(
      body, (E // B,),
      [pl.BlockSpec((B, D), lambda i: (i, 0)),
       pl.BlockSpec((B, 16), lambda i: (i, 0))],
      [pl.BlockSpec((B, dout), lambda i: (i, 0)),
       pl.BlockSpec((B, 64), lambda i: (i, 0))],
      [jax.ShapeDtypeStruct((E, dout), jnp.float32),
       jax.ShapeDtypeStruct((E, 64), jnp.float32)])(rows_l, s)


def _final_logsoftmax(q, den, b2, n, dout):
  Dp = q.shape[2]
  R = 1000

  def body(a_ref, b_ref, dn0, dn1, bias, o_ref):
    o = (a_ref[0][:, :dout] + b_ref[0][:, :dout]) / (
        dn0[0][:, :1] + dn1[0][:, :1] + 1e-16) + bias[...]
    m = jnp.max(o, axis=1, keepdims=True)
    lse = m + jnp.log(jnp.sum(jnp.exp(o - m), axis=1, keepdims=True))
    o_ref[...] = o - lse

  return _tc_call(
      body, (n // R,),
      [pl.BlockSpec((1, R, Dp), lambda i: (0, i, 0)),
       pl.BlockSpec((1, R, Dp), lambda i: (1, i, 0)),
       pl.BlockSpec((1, R, 64), lambda i: (0, i, 0)),
       pl.BlockSpec((1, R, 64), lambda i: (1, i, 0)),
       pl.BlockSpec((1, dout), lambda i: (0, 0))],
      pl.BlockSpec((R, dout), lambda i: (i, 0)),
      jax.ShapeDtypeStruct((n, dout), jnp.float32))(q, q, den, den, b2)


def kernel(x, edge_index, p, Wl1, Wr1, att1, b1, Wl2, Wr2, att2, b2):
  n = x.shape[0]
  heads, ch = att1.shape
  src = edge_index[0].astype(jnp.int32)
  dst = edge_index[1].astype(jnp.int32)

  dout = Wl2.shape[1]

  xl1, xr1 = _mm2(x, Wl1, Wr1)
  rows_l, rows_r = _sc_gather([xl1, xr1], [src, dst])
  s1 = _edge_exp_logits1(rows_l, rows_r, att1.reshape(1, heads * ch),
                         heads, ch)
  msgs = _edge_msg1(rows_l, s1, heads, ch)
  parts = _sc_scatter_many(msgs, dst, n)
  groups, den1 = parts[:-1], parts[-1]

  wl2p = jnp.pad(Wl2, ((0, 0), (0, 128 - dout)))
  wr2p = jnp.pad(Wr2, ((0, 0), (0, 128 - dout)))
  att2p = jnp.pad(att2.reshape(1, -1), ((0, 0), (0, 128 - dout)))
  xl2, xr2 = _combine_elu_mm(groups, den1, b1.reshape(1, -1), wl2p, wr2p,
                             n, heads, ch)

  rows_l2, rows_r2 = _sc_gather([xl2, xr2], [src, dst])
  s2 = _edge_exp_logits2(rows_l2, rows_r2, att2p)
  msg2, s2p = _edge_msg2(rows_l2, s2, dout)
  q, den2 = _sc_scatter_many([msg2, s2p], dst, n)

  return _final_logsoftmax(q, den2, b2.reshape(1, -1), n, dout)
